# baseline (speedup 1.0000x reference)
import functools

import jax
import jax.numpy as jnp
from jax import lax
from jax.experimental import pallas as pl
from jax.experimental.pallas import tpu as pltpu
from jax.experimental.pallas import tpu_sc as plsc

_M = 1000000
_D = 64
_B = 16384
_NC = 2
_NS = 16
_L = 16
_NW = _NC * _NS
_RB = (_M // _NW) // 16 * 16
_TAIL = _M - _NW * _RB
_RLAST = _RB + _TAIL
_WPAD = _RLAST
_CH = 128
_NROWS = _B // _CH + 2
_NBCH = _B // _L

_mesh = plsc.VectorSubcoreMesh(
    core_axis_name="c", subcore_axis_name="s",
    num_cores=_NC, num_subcores=_NS)


@functools.partial(
    pl.kernel,
    out_type=jax.ShapeDtypeStruct((_M, _D), jnp.float32),
    mesh=_mesh,
    scratch_types=[
        pltpu.VMEM((_B,), jnp.int32),
        pltpu.VMEM((_WPAD,), jnp.int32),
        pltpu.VMEM((_NROWS, _CH), jnp.int32),
        pltpu.VMEM((_NROWS, _CH), jnp.int32),
        pltpu.VMEM((_CH, _D), jnp.float32),
        pltpu.SemaphoreType.DMA,
        pltpu.SemaphoreType.DMA,
    ],
    compiler_params=pltpu.CompilerParams(
        needs_layout_passes=False, use_tc_tiling_on_sc=False),
)
def _sc_scatter(in_hbm, idx_hbm, upd_hbm, out_hbm,
                idx_v, win_v, dst_v, src_v, upd_v, copy_sem, g_sem):
    wid = lax.axis_index("s") * _NC + lax.axis_index("c")
    base = wid * _RB
    rsize = jnp.where(wid == _NW - 1, _RLAST, _RB)

    pltpu.sync_copy(idx_hbm, idx_v)
    cdesc = pltpu.async_copy(
        in_hbm.at[pl.ds(base, _RB)], out_hbm.at[pl.ds(base, _RB)], copy_sem)

    @pl.when(wid == _NW - 1)
    def _copy_tail():
        pltpu.sync_copy(in_hbm.at[pl.ds(_NW * _RB, _TAIL)],
                        out_hbm.at[pl.ds(_NW * _RB, _TAIL)])

    iota = lax.broadcasted_iota(jnp.int32, (_L,), 0)
    basev = jnp.full((_L,), base, jnp.int32)
    neg1 = jnp.full((_L,), -1, jnp.int32)

    def init_body(c, carry):
        plsc.store_scatter(win_v, [c * _L + iota], neg1)
        return carry
    lax.fori_loop(0, _WPAD // _L, init_body, 0)

    def sweep_body(c, changed):
        bv = c * _L + iota
        v = plsc.load_gather(idx_v, [bv])
        lv = v - basev
        m = (lv >= 0) & (lv < rsize)
        lvc = jnp.clip(lv, 0, _WPAD - 1)
        cur = plsc.load_gather(win_v, [lvc], mask=m)
        m2 = m & (bv > cur)
        plsc.store_scatter(win_v, [lvc], bv, mask=m2)
        return changed + jnp.max(plsc.all_reduce_population_count(m2))

    def fix_body(_):
        return lax.fori_loop(0, _NBCH, sweep_body, jnp.int32(0))
    lax.while_loop(lambda c: c > 0, fix_body, jnp.int32(1))

    def comp_body(c, carry):
        n_v, fd, fs, have = carry
        bv = c * _L + iota
        v = plsc.load_gather(idx_v, [bv])
        lv = v - basev
        m = (lv >= 0) & (lv < rsize)
        lvc = jnp.clip(lv, 0, _WPAD - 1)
        cur = plsc.load_gather(win_v, [lvc], mask=m)
        keep = m & (cur == bv)
        ki = keep.astype(jnp.int32)
        excl = plsc.cumsum(ki) - ki
        slots = n_v + excl
        rowi = lax.shift_right_logical(slots, 7)
        coli = lax.bitwise_and(slots, _CH - 1)
        plsc.store_scatter(dst_v, [rowi, coli], v, mask=keep)
        plsc.store_scatter(src_v, [rowi, coli], bv, mask=keep)
        cnt = plsc.all_reduce_population_count(keep)
        any_keep = cnt > 0
        onehot = iota == plsc.all_reduce_ffs(keep)
        vbro = jnp.full((_L,), jnp.sum(jnp.where(onehot, v, 0)), jnp.int32)
        bbro = jnp.full((_L,), jnp.sum(jnp.where(onehot, bv, 0)), jnp.int32)
        grab = (have == 0) & any_keep
        fd = jnp.where(grab, vbro, fd)
        fs = jnp.where(grab, bbro, fs)
        have = jnp.where(any_keep, jnp.ones_like(have), have)
        return (n_v + cnt, fd, fs, have)

    zeros = jnp.zeros((_L,), jnp.int32)
    n_v, fd, fs, _have = lax.fori_loop(
        0, _NBCH, comp_body, (zeros, zeros, zeros, zeros))

    def pad_body(k, carry):
        slots = n_v + iota + k * _L
        rowi = lax.shift_right_logical(slots, 7)
        coli = lax.bitwise_and(slots, _CH - 1)
        plsc.store_scatter(dst_v, [rowi, coli], fd)
        plsc.store_scatter(src_v, [rowi, coli], fs)
        return carry
    lax.fori_loop(0, _CH // _L, pad_body, 0)

    cdesc.wait()

    n = jnp.max(n_v)
    nch = (n + _CH - 1) // _CH

    def dma_body(j, carry):
        pltpu.async_copy(upd_hbm.at[src_v.at[j]], upd_v, g_sem).wait()
        pltpu.sync_copy(upd_v, out_hbm.at[dst_v.at[j]])
        return carry
    lax.fori_loop(0, nch, dma_body, 0)


def kernel(inputs, indices, updates):
    idx = indices[:, 0].astype(jnp.int32)
    return _sc_scatter(inputs, idx, updates)


# speedup vs baseline: 1.0020x; 1.0020x over previous
import functools

import jax
import jax.numpy as jnp
from jax import lax
from jax.experimental import pallas as pl
from jax.experimental.pallas import tpu as pltpu
from jax.experimental.pallas import tpu_sc as plsc

_M = 1000000
_D = 64
_B = 16384
_NC = 2
_NS = 16
_L = 16
_NW = _NC * _NS
_RB = (_M // _NW) // 16 * 16
_TAIL = _M - _NW * _RB
_RLAST = _RB + _TAIL
_WPAD = _RLAST
_CH = 128
_NROWS = _B // _CH + 2
_NBCH = _B // _L

_mesh = plsc.VectorSubcoreMesh(
    core_axis_name="c", subcore_axis_name="s",
    num_cores=_NC, num_subcores=_NS)


@functools.partial(
    pl.kernel,
    out_type=jax.ShapeDtypeStruct((_M, _D), jnp.float32),
    mesh=_mesh,
    scratch_types=[
        pltpu.VMEM((_B,), jnp.int32),
        pltpu.VMEM((_WPAD,), jnp.int32),
        pltpu.VMEM((_NROWS, _CH), jnp.int32),
        pltpu.VMEM((_NROWS, _CH), jnp.int32),
        pltpu.VMEM((_CH, _D), jnp.float32),
        pltpu.SemaphoreType.DMA,
        pltpu.SemaphoreType.DMA,
    ],
    compiler_params=pltpu.CompilerParams(
        needs_layout_passes=False, use_tc_tiling_on_sc=False),
)
def _sc_scatter(in_hbm, idx_hbm, upd_hbm, out_hbm,
                idx_v, win_v, dst_v, src_v, upd_v, copy_sem, g_sem):
    wid = lax.axis_index("s") * _NC + lax.axis_index("c")
    base = wid * _RB
    rsize = jnp.where(wid == _NW - 1, _RLAST, _RB)

    pltpu.sync_copy(idx_hbm, idx_v)
    cdesc = pltpu.async_copy(
        in_hbm.at[pl.ds(base, _RB)], out_hbm.at[pl.ds(base, _RB)], copy_sem)

    @pl.when(wid == _NW - 1)
    def _copy_tail():
        pltpu.sync_copy(in_hbm.at[pl.ds(_NW * _RB, _TAIL)],
                        out_hbm.at[pl.ds(_NW * _RB, _TAIL)])

    cdesc.wait()


def kernel(inputs, indices, updates):
    idx = indices[:, 0].astype(jnp.int32)
    return _sc_scatter(inputs, idx, updates)


# speedup vs baseline: 5.7491x; 5.7377x over previous
import functools

import jax
import jax.numpy as jnp
from jax import lax
from jax.experimental import pallas as pl
from jax.experimental.pallas import tpu as pltpu
from jax.experimental.pallas import tpu_sc as plsc

_M = 1000000
_D = 64
_B = 16384
_NC = 2
_NS = 16
_L = 16
_NW = _NC * _NS
_RB = (_M // _NW) // 16 * 16
_TAIL = _M - _NW * _RB
_RLAST = _RB + _TAIL
_WPAD = _RLAST
_CH = 128
_NROWS = _B // _CH + 2
_NBCH = _B // _L
_CHR = 256
_NPAIR = _RB // _CHR // 2
_REM = _RB - _NPAIR * 2 * _CHR

_mesh = plsc.VectorSubcoreMesh(
    core_axis_name="c", subcore_axis_name="s",
    num_cores=_NC, num_subcores=_NS)


@functools.partial(
    pl.kernel,
    out_type=jax.ShapeDtypeStruct((_M, _D), jnp.float32),
    mesh=_mesh,
    scratch_types=[
        pltpu.VMEM((_B,), jnp.int32),
        pltpu.VMEM((_WPAD,), jnp.int32),
        pltpu.VMEM((_NROWS, _CH), jnp.int32),
        pltpu.VMEM((_NROWS, _CH), jnp.int32),
        pltpu.VMEM((_CH, _D), jnp.float32),
        pltpu.VMEM((_CHR, _D), jnp.float32),
        pltpu.VMEM((_CHR, _D), jnp.float32),
        pltpu.SemaphoreType.DMA,
        pltpu.SemaphoreType.DMA,
        pltpu.SemaphoreType.DMA,
        pltpu.SemaphoreType.DMA,
        pltpu.SemaphoreType.DMA,
    ],
    compiler_params=pltpu.CompilerParams(
        needs_layout_passes=False, use_tc_tiling_on_sc=False),
)
def _sc_scatter(in_hbm, idx_hbm, upd_hbm, out_hbm,
                idx_v, win_v, dst_v, src_v, upd_v,
                buf0, buf1, i_sem0, i_sem1, o_sem0, o_sem1, g_sem):
    wid = lax.axis_index("s") * _NC + lax.axis_index("c")
    base = wid * _RB
    rsize = jnp.where(wid == _NW - 1, _RLAST, _RB)

    pltpu.sync_copy(idx_hbm, idx_v)

    iota = lax.broadcasted_iota(jnp.int32, (_L,), 0)
    basev = jnp.full((_L,), base, jnp.int32)
    neg1 = jnp.full((_L,), -1, jnp.int32)

    def init_body(c, carry):
        plsc.store_scatter(win_v, [c * _L + iota], neg1)
        return carry
    lax.fori_loop(0, _WPAD // _L, init_body, 0)

    def sweep_body(c, changed):
        bv = c * _L + iota
        v = plsc.load_gather(idx_v, [bv])
        lv = v - basev
        m = (lv >= 0) & (lv < rsize)
        lvc = jnp.clip(lv, 0, _WPAD - 1)
        cur = plsc.load_gather(win_v, [lvc], mask=m)
        m2 = m & (bv > cur)
        plsc.store_scatter(win_v, [lvc], bv, mask=m2)
        return changed + jnp.max(plsc.all_reduce_population_count(m2))

    def fix_body(_):
        return lax.fori_loop(0, _NBCH, sweep_body, jnp.int32(0))
    lax.while_loop(lambda c: c > 0, fix_body, jnp.int32(1))

    def comp_body(c, carry):
        n_v, fd, fs, have = carry
        bv = c * _L + iota
        v = plsc.load_gather(idx_v, [bv])
        lv = v - basev
        m = (lv >= 0) & (lv < rsize)
        lvc = jnp.clip(lv, 0, _WPAD - 1)
        cur = plsc.load_gather(win_v, [lvc], mask=m)
        keep = m & (cur == bv)
        ki = keep.astype(jnp.int32)
        excl = plsc.cumsum(ki) - ki
        slots = n_v + excl
        rowi = lax.shift_right_logical(slots, 7)
        coli = lax.bitwise_and(slots, _CH - 1)
        plsc.store_scatter(dst_v, [rowi, coli], v, mask=keep)
        plsc.store_scatter(src_v, [rowi, coli], bv, mask=keep)
        cnt = plsc.all_reduce_population_count(keep)
        any_keep = cnt > 0
        onehot = iota == plsc.all_reduce_ffs(keep)
        vbro = jnp.full((_L,), jnp.sum(jnp.where(onehot, v, 0)), jnp.int32)
        bbro = jnp.full((_L,), jnp.sum(jnp.where(onehot, bv, 0)), jnp.int32)
        grab = (have == 0) & any_keep
        fd = jnp.where(grab, vbro, fd)
        fs = jnp.where(grab, bbro, fs)
        have = jnp.where(any_keep, jnp.ones_like(have), have)
        return (n_v + cnt, fd, fs, have)

    zeros = jnp.zeros((_L,), jnp.int32)
    n_v, fd, fs, _have = lax.fori_loop(
        0, _NBCH, comp_body, (zeros, zeros, zeros, zeros))

    def pad_body(k, carry):
        slots = n_v + iota + k * _L
        rowi = lax.shift_right_logical(slots, 7)
        coli = lax.bitwise_and(slots, _CH - 1)
        plsc.store_scatter(dst_v, [rowi, coli], fd)
        plsc.store_scatter(src_v, [rowi, coli], fs)
        return carry
    lax.fori_loop(0, _CH // _L, pad_body, 0)

    def copy_pair(i, carry):
        ca = base + (2 * i) * _CHR
        cb = ca + _CHR

        @pl.when(i > 0)
        def _drain_a():
            pltpu.make_async_copy(buf0, out_hbm.at[pl.ds(ca - 2 * _CHR, _CHR)],
                                  o_sem0).wait()
        pltpu.async_copy(in_hbm.at[pl.ds(ca, _CHR)], buf0, i_sem0).wait()
        pltpu.async_copy(buf0, out_hbm.at[pl.ds(ca, _CHR)], o_sem0)

        @pl.when(i > 0)
        def _drain_b():
            pltpu.make_async_copy(buf1, out_hbm.at[pl.ds(cb - 2 * _CHR, _CHR)],
                                  o_sem1).wait()
        pltpu.async_copy(in_hbm.at[pl.ds(cb, _CHR)], buf1, i_sem1).wait()
        pltpu.async_copy(buf1, out_hbm.at[pl.ds(cb, _CHR)], o_sem1)
        return carry
    lax.fori_loop(0, _NPAIR, copy_pair, 0)
    last = base + (2 * _NPAIR - 1) * _CHR
    pltpu.make_async_copy(buf0, out_hbm.at[pl.ds(last - _CHR, _CHR)],
                          o_sem0).wait()
    pltpu.make_async_copy(buf1, out_hbm.at[pl.ds(last, _CHR)], o_sem1).wait()
    rem = base + 2 * _NPAIR * _CHR
    pltpu.sync_copy(in_hbm.at[pl.ds(rem, _REM)], out_hbm.at[pl.ds(rem, _REM)])

    @pl.when(wid == _NW - 1)
    def _copy_tail():
        pltpu.sync_copy(in_hbm.at[pl.ds(_NW * _RB, _TAIL)],
                        out_hbm.at[pl.ds(_NW * _RB, _TAIL)])

    n = jnp.max(n_v)
    nch = (n + _CH - 1) // _CH

    def dma_body(j, carry):
        pltpu.async_copy(upd_hbm.at[src_v.at[j]], upd_v, g_sem).wait()
        pltpu.sync_copy(upd_v, out_hbm.at[dst_v.at[j]])
        return carry
    lax.fori_loop(0, nch, dma_body, 0)


def kernel(inputs, indices, updates):
    idx = indices[:, 0].astype(jnp.int32)
    return _sc_scatter(inputs, idx, updates)


# speedup vs baseline: 5.9031x; 1.0268x over previous
import functools

import jax
import jax.numpy as jnp
from jax import lax
from jax.experimental import pallas as pl
from jax.experimental.pallas import tpu as pltpu
from jax.experimental.pallas import tpu_sc as plsc

_M = 1000000
_D = 64
_B = 16384
_NC = 2
_NS = 16
_L = 16
_NW = _NC * _NS
_RB = (_M // _NW) // 16 * 16
_TAIL = _M - _NW * _RB
_RLAST = _RB + _TAIL
_WPAD = _RLAST
_CH = 128
_NROWS = _B // _CH + 2
_NBCH = _B // _L
_CHR = 512
_NPAIR = _RB // _CHR // 2
_REM2 = _RB - _NPAIR * 2 * _CHR
_REM = _REM2 - _CHR

_mesh = plsc.VectorSubcoreMesh(
    core_axis_name="c", subcore_axis_name="s",
    num_cores=_NC, num_subcores=_NS)


@functools.partial(
    pl.kernel,
    out_type=jax.ShapeDtypeStruct((_M, _D), jnp.float32),
    mesh=_mesh,
    scratch_types=[
        pltpu.VMEM((_B,), jnp.int32),
        pltpu.VMEM((_NROWS, _CH), jnp.int32),
        pltpu.VMEM((_NROWS, _CH), jnp.int32),
        pltpu.VMEM((_CH, _D), jnp.float32),
        pltpu.SemaphoreType.DMA,
        pltpu.SemaphoreType.DMA,
        pltpu.SemaphoreType.DMA,
        pltpu.SemaphoreType.DMA,
        pltpu.SemaphoreType.DMA,
    ],
    compiler_params=pltpu.CompilerParams(
        needs_layout_passes=False, use_tc_tiling_on_sc=False),
)
def _sc_scatter(in_hbm, idx_hbm, upd_hbm, out_hbm,
                idx_v, dst_v, src_v, upd_v,
                i_sem0, i_sem1, o_sem0, o_sem1, g_sem):
    wid = lax.axis_index("s") * _NC + lax.axis_index("c")
    base = wid * _RB
    rsize = jnp.where(wid == _NW - 1, _RLAST, _RB)

    pltpu.sync_copy(idx_hbm, idx_v)

    iota = lax.broadcasted_iota(jnp.int32, (_L,), 0)
    basev = jnp.full((_L,), base, jnp.int32)
    neg1 = jnp.full((_L,), -1, jnp.int32)

    def dedup_phase(win_v):
        def init_body(c, carry):
            plsc.store_scatter(win_v, [c * _L + iota], neg1)
            return carry
        lax.fori_loop(0, _WPAD // _L, init_body, 0)

        def sweep_body(c, changed):
            bv = c * _L + iota
            v = plsc.load_gather(idx_v, [bv])
            lv = v - basev
            m = (lv >= 0) & (lv < rsize)
            lvc = jnp.clip(lv, 0, _WPAD - 1)
            cur = plsc.load_gather(win_v, [lvc], mask=m)
            m2 = m & (bv > cur)
            plsc.store_scatter(win_v, [lvc], bv, mask=m2)
            return changed + jnp.max(plsc.all_reduce_population_count(m2))

        def fix_body(_):
            return lax.fori_loop(0, _NBCH, sweep_body, jnp.int32(0))
        lax.while_loop(lambda c: c > 0, fix_body, jnp.int32(1))

        def comp_body(c, carry):
            n_v, fd, fs, have = carry
            bv = c * _L + iota
            v = plsc.load_gather(idx_v, [bv])
            lv = v - basev
            m = (lv >= 0) & (lv < rsize)
            lvc = jnp.clip(lv, 0, _WPAD - 1)
            cur = plsc.load_gather(win_v, [lvc], mask=m)
            keep = m & (cur == bv)
            ki = keep.astype(jnp.int32)
            excl = plsc.cumsum(ki) - ki
            slots = n_v + excl
            rowi = lax.shift_right_logical(slots, 7)
            coli = lax.bitwise_and(slots, _CH - 1)
            plsc.store_scatter(dst_v, [rowi, coli], v, mask=keep)
            plsc.store_scatter(src_v, [rowi, coli], bv, mask=keep)
            cnt = plsc.all_reduce_population_count(keep)
            any_keep = cnt > 0
            onehot = iota == plsc.all_reduce_ffs(keep)
            vbro = jnp.full((_L,), jnp.sum(jnp.where(onehot, v, 0)), jnp.int32)
            bbro = jnp.full((_L,), jnp.sum(jnp.where(onehot, bv, 0)), jnp.int32)
            grab = (have == 0) & any_keep
            fd = jnp.where(grab, vbro, fd)
            fs = jnp.where(grab, bbro, fs)
            have = jnp.where(any_keep, jnp.ones_like(have), have)
            return (n_v + cnt, fd, fs, have)

        zeros = jnp.zeros((_L,), jnp.int32)
        n_v, fd, fs, _have = lax.fori_loop(
            0, _NBCH, comp_body, (zeros, zeros, zeros, zeros))

        def pad_body(k, carry):
            slots = n_v + iota + k * _L
            rowi = lax.shift_right_logical(slots, 7)
            coli = lax.bitwise_and(slots, _CH - 1)
            plsc.store_scatter(dst_v, [rowi, coli], fd)
            plsc.store_scatter(src_v, [rowi, coli], fs)
            return carry
        lax.fori_loop(0, _CH // _L, pad_body, 0)
        return n_v

    n_v = pl.run_scoped(dedup_phase, pltpu.VMEM((_WPAD,), jnp.int32))

    def copy_phase(buf0, buf1):
        def copy_pair(i, carry):
            ca = base + (2 * i) * _CHR
            cb = ca + _CHR

            @pl.when(i > 0)
            def _drain_a():
                pltpu.make_async_copy(
                    buf0, out_hbm.at[pl.ds(ca - 2 * _CHR, _CHR)], o_sem0).wait()
            pltpu.async_copy(in_hbm.at[pl.ds(ca, _CHR)], buf0, i_sem0).wait()
            pltpu.async_copy(buf0, out_hbm.at[pl.ds(ca, _CHR)], o_sem0)

            @pl.when(i > 0)
            def _drain_b():
                pltpu.make_async_copy(
                    buf1, out_hbm.at[pl.ds(cb - 2 * _CHR, _CHR)], o_sem1).wait()
            pltpu.async_copy(in_hbm.at[pl.ds(cb, _CHR)], buf1, i_sem1).wait()
            pltpu.async_copy(buf1, out_hbm.at[pl.ds(cb, _CHR)], o_sem1)
            return carry
        lax.fori_loop(0, _NPAIR, copy_pair, 0)
        last = base + (2 * _NPAIR - 1) * _CHR
        pltpu.make_async_copy(buf0, out_hbm.at[pl.ds(last - _CHR, _CHR)],
                              o_sem0).wait()
        ex = base + 2 * _NPAIR * _CHR
        pltpu.async_copy(in_hbm.at[pl.ds(ex, _CHR)], buf0, i_sem0).wait()
        pltpu.async_copy(buf0, out_hbm.at[pl.ds(ex, _CHR)], o_sem0)
        pltpu.make_async_copy(buf1, out_hbm.at[pl.ds(last, _CHR)], o_sem1).wait()
        pltpu.make_async_copy(buf0, out_hbm.at[pl.ds(ex, _CHR)], o_sem0).wait()

    pl.run_scoped(copy_phase,
                  pltpu.VMEM((_CHR, _D), jnp.float32),
                  pltpu.VMEM((_CHR, _D), jnp.float32))
    rem = base + 2 * _NPAIR * _CHR + _CHR
    pltpu.sync_copy(in_hbm.at[pl.ds(rem, _REM)], out_hbm.at[pl.ds(rem, _REM)])

    @pl.when(wid == _NW - 1)
    def _copy_tail():
        pltpu.sync_copy(in_hbm.at[pl.ds(_NW * _RB, _TAIL)],
                        out_hbm.at[pl.ds(_NW * _RB, _TAIL)])

    n = jnp.max(n_v)
    nch = (n + _CH - 1) // _CH

    def dma_body(j, carry):
        pltpu.async_copy(upd_hbm.at[src_v.at[j]], upd_v, g_sem).wait()
        pltpu.sync_copy(upd_v, out_hbm.at[dst_v.at[j]])
        return carry
    lax.fori_loop(0, nch, dma_body, 0)


def kernel(inputs, indices, updates):
    idx = indices[:, 0].astype(jnp.int32)
    return _sc_scatter(inputs, idx, updates)
